# baseline (speedup 1.0000x reference)
import functools

import jax
import jax.numpy as jnp
from jax import lax
from jax.experimental import pallas as pl
from jax.experimental.pallas import tpu as pltpu
from jax.experimental.pallas import tpu_sc as plsc

_N = 10000
_E = 320000
_D = 128
_G = 64
_NC = 2
_NS = 16
_NW = _NC * _NS
_CH = 128
_N_ACC = 10112
_RPT = _N_ACC // _NS

_IB = 16
_NBLK = 5
_NCHUNKS = _IB * _NBLK
_EPT = _NCHUNKS * _CH
_EPAD = _NW * _EPT


def _agg_call(h, src_p, dst_p):
    mesh = plsc.VectorSubcoreMesh(core_axis_name="c", subcore_axis_name="s")

    @functools.partial(
        pl.kernel,
        out_type=jax.ShapeDtypeStruct((_NC * _N_ACC, _D), jnp.float32),
        mesh=mesh,
        scratch_types=[
            pltpu.VMEM((2, _IB, _CH), jnp.int32),
            pltpu.VMEM((2, _IB, _CH), jnp.int32),
            pltpu.VMEM((2, _CH, _D), jnp.float32),
            pltpu.VMEM_SHARED((_N_ACC, _D), jnp.float32),
            pltpu.SemaphoreType.DMA,
            pltpu.SemaphoreType.DMA,
            pltpu.SemaphoreType.DMA,
            pltpu.SemaphoreType.DMA,
        ],
    )
    def k(h_hbm, src_hbm, dst_hbm, out_hbm, srci, dsti, rows, acc,
          sem0, sem1, isem0, isem1):
        c = lax.axis_index("c")
        s = lax.axis_index("s")
        wid = s * _NC + c
        isems = (isem0, isem1)

        def idx_refill(blk, sl):
            sem = isems[sl]
            pltpu.async_copy(src_hbm.at[wid, pl.ds(blk * _IB, _IB)],
                             srci.at[sl], sem)
            pltpu.async_copy(dst_hbm.at[wid, pl.ds(blk * _IB, _IB)],
                             dsti.at[sl], sem)

        def idx_wait(sl):
            sem = isems[sl]
            pltpu.make_async_copy(src_hbm.at[wid, pl.ds(0, _IB)],
                                  srci.at[sl], sem).wait()
            pltpu.make_async_copy(dst_hbm.at[wid, pl.ds(0, _IB)],
                                  dsti.at[sl], sem).wait()

        idx_refill(0, 0)

        zrow = jnp.zeros((16,), jnp.float32)

        def zbody(i, carry):
            for u in range(_D // 16):
                rows[0, i, pl.ds(u * 16, 16)] = zrow
            return carry

        lax.fori_loop(0, _CH, zbody, 0)
        base = s * _RPT
        for q in range(_RPT // _CH):
            pltpu.sync_copy(rows.at[0], acc.at[pl.ds(base + q * _CH, _CH)])
        rem = _RPT % _CH
        if rem:
            pltpu.sync_copy(rows.at[0, pl.ds(0, rem)],
                            acc.at[pl.ds(base + (_RPT // _CH) * _CH, rem)])
        plsc.subcore_barrier()

        def gather(sl, j, b, sem):
            return pltpu.async_copy(h_hbm.at[srci.at[sl, j]], rows.at[b], sem)

        def gwait(sl, b, sem):
            pltpu.make_async_copy(h_hbm.at[srci.at[sl, 0]], rows.at[b],
                                  sem).wait()

        def scatter(sl, j, b):
            pltpu.sync_copy(rows.at[b], acc.at[dsti.at[sl, j]], add=True)

        for blk in range(_NBLK):
            sl = blk % 2
            idx_wait(sl)
            if blk + 1 < _NBLK:
                idx_refill(blk + 1, 1 - sl)
            gather(sl, 0, 0, sem0)

            def body(p, carry, sl=sl):
                j0 = 2 * p
                gather(sl, j0 + 1, 1, sem1)
                gwait(sl, 0, sem0)
                scatter(sl, j0, 0)
                gather(sl, j0 + 2, 0, sem0)
                gwait(sl, 1, sem1)
                scatter(sl, j0 + 1, 1)
                return carry

            lax.fori_loop(0, _IB // 2 - 1, body, 0)
            gather(sl, _IB - 1, 1, sem1)
            gwait(sl, 0, sem0)
            scatter(sl, _IB - 2, 0)
            gwait(sl, 1, sem1)
            scatter(sl, _IB - 1, 1)
        plsc.subcore_barrier()

        ob = c * _N_ACC + base
        pltpu.sync_copy(acc.at[pl.ds(base, _RPT)], out_hbm.at[pl.ds(ob, _RPT)])

    return k(h, src_p, dst_p)


def _mlp_body(h_ref, a0_ref, a1_ref, w1_ref, b1_ref, w2_ref, b2_ref,
              g_ref, be_ref, o_ref):
    z = h_ref[...] + a0_ref[...] + a1_ref[...]
    z = jnp.maximum(
        jnp.dot(z, w1_ref[...], preferred_element_type=jnp.float32)
        + b1_ref[...], 0.0)
    z = jnp.maximum(
        jnp.dot(z, w2_ref[...], preferred_element_type=jnp.float32)
        + b2_ref[...], 0.0)
    m = jnp.mean(z, axis=0, keepdims=True)
    v = jnp.mean((z - m) ** 2, axis=0, keepdims=True)
    o_ref[...] = (z - m) / jnp.sqrt(v + 1e-5) * g_ref[...] + be_ref[...]


def _mlp_call(h, a0, a1, w1, b1, w2, b2, g, be):
    return pl.pallas_call(
        _mlp_body,
        out_shape=jax.ShapeDtypeStruct((_N, _D), jnp.float32),
    )(h, a0, a1, w1, b1.reshape(1, _D), w2, b2.reshape(1, _D),
      g.reshape(1, _D), be.reshape(1, _D))


def _final_body(h_ref, bidx_ref, lw1_ref, lb1_ref, lw2_ref, lb2_ref, o_ref):
    h = h_ref[...]
    bidx = bidx_ref[...]
    rid = lax.broadcasted_iota(jnp.int32, (_G, 1), 0)
    neg = jnp.float32(-jnp.inf)

    def body(gi, carry):
        col = jnp.max(jnp.where(bidx == gi, h, neg), axis=0, keepdims=True)
        return jnp.where(rid == gi, col, carry)

    pooled = lax.fori_loop(0, _G, body,
                           jnp.full((_G, _D), neg, jnp.float32))
    r = jnp.maximum(
        jnp.dot(pooled, lw1_ref[...], preferred_element_type=jnp.float32)
        + lb1_ref[...], 0.0)
    o_ref[...] = (jnp.dot(r, lw2_ref[...], preferred_element_type=jnp.float32)
                  + lb2_ref[...])


def _final_call(h, batch, lw1, lb1, lw2, lb2):
    c = lw2.shape[1]
    lw2p = jnp.zeros((_D, _D), jnp.float32).at[:, :c].set(lw2)
    lb2p = jnp.zeros((1, _D), jnp.float32).at[:, :c].set(lb2.reshape(1, c))
    out = pl.pallas_call(
        _final_body,
        out_shape=jax.ShapeDtypeStruct((_G, _D), jnp.float32),
    )(h, batch.reshape(_N, 1), lw1, lb1.reshape(1, _D), lw2p, lb2p)
    return out[:, :c]


def kernel(x, edge_index, batch,
           W1_0, b1_0, W2_0, b2_0, g_0, be_0,
           W1_1, b1_1, W2_1, b2_1, g_1, be_1,
           W1_2, b1_2, W2_2, b2_2, g_2, be_2,
           lw1, lb1, lw2, lb2):
    src = edge_index[0]
    dst = edge_index[1]
    npad = _EPAD - _E
    src_p = jnp.concatenate(
        [src, jnp.zeros((npad,), jnp.int32)]).reshape(_NW, _NCHUNKS, _CH)
    dst_p = jnp.concatenate(
        [dst, jnp.full((npad,), _N, jnp.int32)]).reshape(_NW, _NCHUNKS, _CH)

    layers = [(W1_0, b1_0, W2_0, b2_0, g_0, be_0),
              (W1_1, b1_1, W2_1, b2_1, g_1, be_1),
              (W1_2, b1_2, W2_2, b2_2, g_2, be_2)]
    h = x
    for (w1, b1, w2, b2, g, be) in layers:
        agg = _agg_call(h, src_p, dst_p)
        a0 = agg[:_N]
        a1 = agg[_N_ACC:_N_ACC + _N]
        h = _mlp_call(h, a0, a1, w1, b1, w2, b2, g, be)
    return _final_call(h, batch, lw1, lb1, lw2, lb2)


# speedup vs baseline: 1.0903x; 1.0903x over previous
import functools

import jax
import jax.numpy as jnp
from jax import lax
from jax.experimental import pallas as pl
from jax.experimental.pallas import tpu as pltpu
from jax.experimental.pallas import tpu_sc as plsc

_N = 10000
_E = 320000
_D = 128
_G = 64
_NC = 2
_NS = 16
_CH = 128
_N_ACC = 10112
_RPT = _N_ACC // _NS

_IB = 16
_B0 = 8
_B1 = 2
_K0 = _IB * _B0
_K1 = _IB * _B1
_E0 = _NS * _K0 * _CH
_E1 = _NS * _K1 * _CH
_EPAD = _E0 + _E1


def _agg_call(h, s0src, s0dst, s1src, s1dst):
    mesh = plsc.VectorSubcoreMesh(core_axis_name="c", subcore_axis_name="s")

    @functools.partial(
        pl.kernel,
        out_type=jax.ShapeDtypeStruct((_NC, _N_ACC, _D), jnp.float32),
        mesh=mesh,
        scratch_types=[
            pltpu.VMEM((2, _IB, _CH), jnp.int32),
            pltpu.VMEM((2, _IB, _CH), jnp.int32),
            pltpu.VMEM((2, _CH, _D), jnp.float32),
            pltpu.VMEM_SHARED((_N_ACC, _D), jnp.float32),
            pltpu.SemaphoreType.DMA,
            pltpu.SemaphoreType.DMA,
            pltpu.SemaphoreType.DMA,
            pltpu.SemaphoreType.DMA,
        ],
    )
    def k(h_hbm, s0s_hbm, s0d_hbm, s1s_hbm, s1d_hbm, out_hbm, srci, dsti,
          rows, acc, sem0, sem1, isem0, isem1):
        c = lax.axis_index("c")
        s = lax.axis_index("s")
        isems = (isem0, isem1)

        zrow = jnp.zeros((16,), jnp.float32)

        def zbody(i, carry):
            for u in range(_D // 16):
                rows[0, i, pl.ds(u * 16, 16)] = zrow
            return carry

        lax.fori_loop(0, _CH, zbody, 0)
        base = s * _RPT
        chunks = [(q * _CH, _CH) for q in range(_RPT // _CH)]
        if _RPT % _CH:
            chunks.append(((_RPT // _CH) * _CH, _RPT % _CH))
        for off, sz in chunks:
            pltpu.sync_copy(rows.at[0, pl.ds(0, sz)],
                            acc.at[pl.ds(base + off, sz)])
        plsc.subcore_barrier()

        def run(src_hbm, dst_hbm, nblk):
            def idx_refill(blk, sl):
                sem = isems[sl]
                pltpu.async_copy(src_hbm.at[s, pl.ds(blk * _IB, _IB)],
                                 srci.at[sl], sem)
                pltpu.async_copy(dst_hbm.at[s, pl.ds(blk * _IB, _IB)],
                                 dsti.at[sl], sem)

            def idx_wait(sl):
                sem = isems[sl]
                pltpu.make_async_copy(src_hbm.at[s, pl.ds(0, _IB)],
                                      srci.at[sl], sem).wait()
                pltpu.make_async_copy(dst_hbm.at[s, pl.ds(0, _IB)],
                                      dsti.at[sl], sem).wait()

            def gather(sl, j, b, sem):
                return pltpu.async_copy(h_hbm.at[srci.at[sl, j]],
                                        rows.at[b], sem)

            def gwait(b, sem):
                pltpu.make_async_copy(h_hbm.at[srci.at[0, 0]], rows.at[b],
                                      sem).wait()

            def scatter(sl, j, b):
                pltpu.sync_copy(rows.at[b], acc.at[dsti.at[sl, j]], add=True)

            idx_refill(0, 0)
            for blk in range(nblk):
                sl = blk % 2
                idx_wait(sl)
                if blk + 1 < nblk:
                    idx_refill(blk + 1, 1 - sl)
                gather(sl, 0, 0, sem0)

                def body(p, carry, sl=sl):
                    j0 = 2 * p
                    gather(sl, j0 + 1, 1, sem1)
                    gwait(0, sem0)
                    scatter(sl, j0, 0)
                    gather(sl, j0 + 2, 0, sem0)
                    gwait(1, sem1)
                    scatter(sl, j0 + 1, 1)
                    return carry

                lax.fori_loop(0, _IB // 2 - 1, body, 0)
                gather(sl, _IB - 1, 1, sem1)
                gwait(0, sem0)
                scatter(sl, _IB - 2, 0)
                gwait(1, sem1)
                scatter(sl, _IB - 1, 1)

        @pl.when(c == 0)
        def _():
            run(s0s_hbm, s0d_hbm, _B0)

        @pl.when(c == 1)
        def _():
            run(s1s_hbm, s1d_hbm, _B1)

        plsc.subcore_barrier()

        pltpu.sync_copy(acc.at[pl.ds(base, _RPT)],
                        out_hbm.at[c, pl.ds(base, _RPT)])

    return k(h, s0src, s0dst, s1src, s1dst)


def _mlp_body(h_ref, a0_ref, a1_ref, w1_ref, b1_ref, w2_ref, b2_ref,
              g_ref, be_ref, o_ref):
    z = h_ref[...] + a0_ref[...] + a1_ref[...]
    z = jnp.maximum(
        jnp.dot(z, w1_ref[...], preferred_element_type=jnp.float32)
        + b1_ref[...], 0.0)
    z = jnp.maximum(
        jnp.dot(z, w2_ref[...], preferred_element_type=jnp.float32)
        + b2_ref[...], 0.0)
    m = jnp.mean(z, axis=0, keepdims=True)
    v = jnp.mean((z - m) ** 2, axis=0, keepdims=True)
    o_ref[...] = (z - m) / jnp.sqrt(v + 1e-5) * g_ref[...] + be_ref[...]


def _mlp_call(h, a0, a1, w1, b1, w2, b2, g, be):
    return pl.pallas_call(
        _mlp_body,
        out_shape=jax.ShapeDtypeStruct((_N, _D), jnp.float32),
    )(h, a0, a1, w1, b1.reshape(1, _D), w2, b2.reshape(1, _D),
      g.reshape(1, _D), be.reshape(1, _D))


def _final_body(h_ref, bidx_ref, lw1_ref, lb1_ref, lw2_ref, lb2_ref, o_ref):
    h = h_ref[...]
    bidx = bidx_ref[...]
    rid = lax.broadcasted_iota(jnp.int32, (_G, 1), 0)
    neg = jnp.float32(-jnp.inf)

    def body(gi, carry):
        col = jnp.max(jnp.where(bidx == gi, h, neg), axis=0, keepdims=True)
        return jnp.where(rid == gi, col, carry)

    pooled = lax.fori_loop(0, _G, body,
                           jnp.full((_G, _D), neg, jnp.float32))
    r = jnp.maximum(
        jnp.dot(pooled, lw1_ref[...], preferred_element_type=jnp.float32)
        + lb1_ref[...], 0.0)
    o_ref[...] = (jnp.dot(r, lw2_ref[...], preferred_element_type=jnp.float32)
                  + lb2_ref[...])


def _final_call(h, batch, lw1, lb1, lw2, lb2):
    c = lw2.shape[1]
    lw2p = jnp.zeros((_D, _D), jnp.float32).at[:, :c].set(lw2)
    lb2p = jnp.zeros((1, _D), jnp.float32).at[:, :c].set(lb2.reshape(1, c))
    out = pl.pallas_call(
        _final_body,
        out_shape=jax.ShapeDtypeStruct((_G, _D), jnp.float32),
    )(h, batch.reshape(_N, 1), lw1, lb1.reshape(1, _D), lw2p, lb2p)
    return out[:, :c]


def kernel(x, edge_index, batch,
           W1_0, b1_0, W2_0, b2_0, g_0, be_0,
           W1_1, b1_1, W2_1, b2_1, g_1, be_1,
           W1_2, b1_2, W2_2, b2_2, g_2, be_2,
           lw1, lb1, lw2, lb2):
    src = edge_index[0]
    dst = edge_index[1]
    npad = _EPAD - _E
    src_p = jnp.concatenate([src, jnp.zeros((npad,), jnp.int32)])
    dst_p = jnp.concatenate([dst, jnp.full((npad,), _N, jnp.int32)])
    s0src = src_p[:_E0].reshape(_NS, _K0, _CH)
    s0dst = dst_p[:_E0].reshape(_NS, _K0, _CH)
    s1src = src_p[_E0:].reshape(_NS, _K1, _CH)
    s1dst = dst_p[_E0:].reshape(_NS, _K1, _CH)

    layers = [(W1_0, b1_0, W2_0, b2_0, g_0, be_0),
              (W1_1, b1_1, W2_1, b2_1, g_1, be_1),
              (W1_2, b1_2, W2_2, b2_2, g_2, be_2)]
    h = x
    for (w1, b1, w2, b2, g, be) in layers:
        agg = _agg_call(h, s0src, s0dst, s1src, s1dst)
        a0 = agg[0, :_N]
        a1 = agg[1, :_N]
        h = _mlp_call(h, a0, a1, w1, b1, w2, b2, g, be)
    return _final_call(h, batch, lw1, lb1, lw2, lb2)
